# baseline (speedup 1.0000x reference)
import functools

import jax
import jax.numpy as jnp
from jax import lax
from jax.experimental import pallas as pl
from jax.experimental.pallas import tpu as pltpu
from jax.experimental.pallas import tpu_sc as plsc

N_NODES = 10000
N_EDGES = 320000
DIM = 128

NC = 2
NS = 16
NW = NC * NS
CH = 128
NG = 2
G = 2 * -(-N_EDGES // (NW * CH * NG * 2)) * NG // NG
K = G * NG
EPW = K * CH
EPAD = EPW * NW
ZR = -(-(N_NODES + 1) // (NS * 8)) * 8
A_ROWS = ZR * NS


def _sc_segment_sum(x, src, dst, zrows):
    mesh = plsc.VectorSubcoreMesh(core_axis_name="c", subcore_axis_name="s")

    @functools.partial(
        pl.kernel,
        mesh=mesh,
        out_type=jax.ShapeDtypeStruct((NC, A_ROWS, DIM), jnp.float32),
        scratch_types=[
            pltpu.VMEM((G, CH), jnp.int32),
            pltpu.VMEM((G, CH), jnp.int32),
            pltpu.VMEM((2, CH, DIM), jnp.float32),
            pltpu.VMEM_SHARED((A_ROWS, DIM), jnp.float32),
            pltpu.SemaphoreType.DMA,
            pltpu.SemaphoreType.DMA,
        ],
    )
    def body(x_hbm, src_hbm, dst_hbm, z_hbm, out_hbm, src_v, dst_v, rows_v, acc,
             gsem0, gsem1):
        c = lax.axis_index("c")
        s = lax.axis_index("s")
        wid = s * NC + c

        pltpu.sync_copy(z_hbm, acc.at[pl.ds(s * ZR, ZR)])
        plsc.subcore_barrier()

        for ng in range(NG):
            pltpu.sync_copy(src_hbm.at[wid, pl.ds(ng * G, G)], src_v)
            pltpu.sync_copy(dst_hbm.at[wid, pl.ds(ng * G, G)], dst_v)

            pltpu.async_copy(x_hbm.at[src_v.at[0]], rows_v.at[0], gsem0)

            def step(jj, _):
                j = 2 * jj
                pltpu.make_async_copy(x_hbm.at[src_v.at[j]], rows_v.at[0],
                                      gsem0).wait()
                pltpu.async_copy(x_hbm.at[src_v.at[j + 1]], rows_v.at[1], gsem1)
                pltpu.sync_copy(rows_v.at[0], acc.at[dst_v.at[j]], add=True)
                pltpu.make_async_copy(x_hbm.at[src_v.at[j + 1]], rows_v.at[1],
                                      gsem1).wait()
                pltpu.async_copy(x_hbm.at[src_v.at[j + 2]], rows_v.at[0], gsem0)
                pltpu.sync_copy(rows_v.at[1], acc.at[dst_v.at[j + 1]], add=True)
                return _

            lax.fori_loop(0, G // 2 - 1, step, None)
            j = G - 2
            pltpu.make_async_copy(x_hbm.at[src_v.at[j]], rows_v.at[0],
                                  gsem0).wait()
            pltpu.async_copy(x_hbm.at[src_v.at[j + 1]], rows_v.at[1], gsem1)
            pltpu.sync_copy(rows_v.at[0], acc.at[dst_v.at[j]], add=True)
            pltpu.make_async_copy(x_hbm.at[src_v.at[j + 1]], rows_v.at[1],
                                  gsem1).wait()
            pltpu.sync_copy(rows_v.at[1], acc.at[dst_v.at[j + 1]], add=True)
        plsc.subcore_barrier()
        pltpu.sync_copy(acc.at[pl.ds(s * ZR, ZR)],
                        out_hbm.at[c, pl.ds(s * ZR, ZR)])

    return body(x, src, dst, zrows)


def _tc_finish(x, a, w1t, w2t):
    R = 1000

    def body(x_ref, a0_ref, a1_ref, w1t_ref, w2t_ref, o_ref):
        sp = jnp.dot(x_ref[...], w1t_ref[...],
                     preferred_element_type=jnp.float32,
                     precision=lax.Precision.HIGHEST)
        np_ = jnp.dot(a0_ref[0] + a1_ref[0], w2t_ref[...],
                      preferred_element_type=jnp.float32,
                      precision=lax.Precision.HIGHEST)
        o_ref[...] = jnp.maximum(sp + np_, 0.0)

    return pl.pallas_call(
        body,
        grid=(N_NODES // R,),
        in_specs=[
            pl.BlockSpec((R, DIM), lambda i: (i, 0)),
            pl.BlockSpec((1, R, DIM), lambda i: (0, i, 0)),
            pl.BlockSpec((1, R, DIM), lambda i: (1, i, 0)),
            pl.BlockSpec((DIM, DIM), lambda i: (0, 0)),
            pl.BlockSpec((DIM, DIM), lambda i: (0, 0)),
        ],
        out_specs=pl.BlockSpec((R, DIM), lambda i: (i, 0)),
        out_shape=jax.ShapeDtypeStruct((N_NODES, DIM), jnp.float32),
    )(x, a, a, w1t, w2t)


def kernel(x, edge_index, W1, W2):
    src = edge_index[0].astype(jnp.int32)
    dst = edge_index[1].astype(jnp.int32)
    pad = EPAD - N_EDGES
    src_p = jnp.concatenate([src, jnp.zeros((pad,), jnp.int32)]).reshape(NW, K, CH)
    dst_p = jnp.concatenate([dst, jnp.full((pad,), N_NODES, jnp.int32)]).reshape(NW, K, CH)
    zrows = jnp.zeros((ZR, DIM), jnp.float32)
    a = _sc_segment_sum(x, src_p, dst_p, zrows)
    return _tc_finish(x, a, W1.T, W2.T)


# speedup vs baseline: 1.4402x; 1.4402x over previous
import functools

import jax
import jax.numpy as jnp
from jax import lax
from jax.experimental import pallas as pl
from jax.experimental.pallas import tpu as pltpu
from jax.experimental.pallas import tpu_sc as plsc

N_NODES = 10000
N_EDGES = 320000
DIM = 128

NC = 2
NS = 16
NW = NC * NS
CH = 128
K = -(-N_EDGES // (NW * CH))
EPW = K * CH
EPAD = EPW * NW
ZR = -(-(N_NODES + 1) // (NS * 8)) * 8
A_ROWS = ZR * NS


def _sc_segment_sum(x, src, dst, zrows):
    mesh = plsc.VectorSubcoreMesh(core_axis_name="c", subcore_axis_name="s")

    @functools.partial(
        pl.kernel,
        mesh=mesh,
        out_type=jax.ShapeDtypeStruct((NC, A_ROWS, DIM), jnp.float32),
        scratch_types=[
            pltpu.VMEM((K, CH), jnp.int32),
            pltpu.VMEM((K, CH), jnp.int32),
            pltpu.VMEM((CH, DIM), jnp.float32),
            pltpu.VMEM_SHARED((A_ROWS, DIM), jnp.float32),
        ],
    )
    def body(x_hbm, src_hbm, dst_hbm, z_hbm, out_hbm, src_v, dst_v, rows_v, acc):
        c = lax.axis_index("c")
        s = lax.axis_index("s")
        wid = s * NC + c

        pltpu.sync_copy(z_hbm, acc.at[pl.ds(s * ZR, ZR)])
        pltpu.sync_copy(src_hbm.at[wid], src_v)
        pltpu.sync_copy(dst_hbm.at[wid], dst_v)
        plsc.subcore_barrier()

        def step(j, _):
            pltpu.sync_copy(x_hbm.at[src_v.at[j]], rows_v)
            pltpu.sync_copy(rows_v, acc.at[dst_v.at[j]], add=True)
            return _

        lax.fori_loop(0, K, step, None)
        plsc.subcore_barrier()
        pltpu.sync_copy(acc.at[pl.ds(s * ZR, ZR)],
                        out_hbm.at[c, pl.ds(s * ZR, ZR)])

    return body(x, src, dst, zrows)


def _tc_finish(x, a, w1t, w2t):
    R = 1000

    def body(x_ref, a0_ref, a1_ref, w1t_ref, w2t_ref, o_ref):
        sp = jnp.dot(x_ref[...], w1t_ref[...],
                     preferred_element_type=jnp.float32,
                     precision=lax.Precision.HIGHEST)
        np_ = jnp.dot(a0_ref[0] + a1_ref[0], w2t_ref[...],
                      preferred_element_type=jnp.float32,
                      precision=lax.Precision.HIGHEST)
        o_ref[...] = jnp.maximum(sp + np_, 0.0)

    return pl.pallas_call(
        body,
        grid=(N_NODES // R,),
        in_specs=[
            pl.BlockSpec((R, DIM), lambda i: (i, 0)),
            pl.BlockSpec((1, R, DIM), lambda i: (0, i, 0)),
            pl.BlockSpec((1, R, DIM), lambda i: (1, i, 0)),
            pl.BlockSpec((DIM, DIM), lambda i: (0, 0)),
            pl.BlockSpec((DIM, DIM), lambda i: (0, 0)),
        ],
        out_specs=pl.BlockSpec((R, DIM), lambda i: (i, 0)),
        out_shape=jax.ShapeDtypeStruct((N_NODES, DIM), jnp.float32),
    )(x, a, a, w1t, w2t)


def kernel(x, edge_index, W1, W2):
    src = edge_index[0].astype(jnp.int32)
    dst = edge_index[1].astype(jnp.int32)
    pad = EPAD - N_EDGES
    src_p = jnp.concatenate([src, jnp.zeros((pad,), jnp.int32)]).reshape(NW, K, CH)
    dst_p = jnp.concatenate([dst, jnp.full((pad,), N_NODES, jnp.int32)]).reshape(NW, K, CH)
    zrows = jnp.zeros((ZR, DIM), jnp.float32)
    a = _sc_segment_sum(x, src_p, dst_p, zrows)
    return _tc_finish(x, a, W1.T, W2.T)


# speedup vs baseline: 1.9865x; 1.3793x over previous
import functools

import jax
import jax.numpy as jnp
from jax import lax
from jax.experimental import pallas as pl
from jax.experimental.pallas import tpu as pltpu
from jax.experimental.pallas import tpu_sc as plsc

N_NODES = 10000
N_EDGES = 320000
DIM = 128

NC = 2
NS = 16
NW = NC * NS
CH = 128
K0 = 105
K1 = 52
E0 = NS * K0 * CH
E1 = NS * K1 * CH
EPAD = E0 + E1
ZR = -(-(N_NODES + 1) // (NS * 8)) * 8
A_ROWS = ZR * NS


def _sc_segment_sum(x, src, dst, zrows):
    mesh = plsc.VectorSubcoreMesh(core_axis_name="c", subcore_axis_name="s")

    @functools.partial(
        pl.kernel,
        mesh=mesh,
        out_type=jax.ShapeDtypeStruct((NC, A_ROWS, DIM), jnp.float32),
        scratch_types=[
            pltpu.VMEM((K0, CH), jnp.int32),
            pltpu.VMEM((K0, CH), jnp.int32),
            pltpu.VMEM((CH, DIM), jnp.float32),
            pltpu.VMEM_SHARED((A_ROWS, DIM), jnp.float32),
        ],
    )
    def body(x_hbm, src_hbm, dst_hbm, z_hbm, out_hbm, src_v, dst_v, rows_v, acc):
        c = lax.axis_index("c")
        s = lax.axis_index("s")
        wid = s * NC + c

        pltpu.sync_copy(z_hbm, acc.at[pl.ds(s * ZR, ZR)])
        pltpu.sync_copy(src_hbm.at[wid], src_v)
        pltpu.sync_copy(dst_hbm.at[wid], dst_v)
        plsc.subcore_barrier()

        def step(j, _):
            pltpu.sync_copy(x_hbm.at[src_v.at[j]], rows_v)
            pltpu.sync_copy(rows_v, acc.at[dst_v.at[j]], add=True)
            return _

        kc = jnp.where(c == 0, K0, K1)
        lax.fori_loop(0, kc, step, None)
        plsc.subcore_barrier()
        pltpu.sync_copy(acc.at[pl.ds(s * ZR, ZR)],
                        out_hbm.at[c, pl.ds(s * ZR, ZR)])

    return body(x, src, dst, zrows)


def _tc_finish(x, a, w1t, w2t):
    R = 1000

    def body(x_ref, a0_ref, a1_ref, w1t_ref, w2t_ref, o_ref):
        sp = jnp.dot(x_ref[...], w1t_ref[...],
                     preferred_element_type=jnp.float32,
                     precision=lax.Precision.HIGHEST)
        np_ = jnp.dot(a0_ref[0] + a1_ref[0], w2t_ref[...],
                      preferred_element_type=jnp.float32,
                      precision=lax.Precision.HIGHEST)
        o_ref[...] = jnp.maximum(sp + np_, 0.0)

    return pl.pallas_call(
        body,
        grid=(N_NODES // R,),
        in_specs=[
            pl.BlockSpec((R, DIM), lambda i: (i, 0)),
            pl.BlockSpec((1, R, DIM), lambda i: (0, i, 0)),
            pl.BlockSpec((1, R, DIM), lambda i: (1, i, 0)),
            pl.BlockSpec((DIM, DIM), lambda i: (0, 0)),
            pl.BlockSpec((DIM, DIM), lambda i: (0, 0)),
        ],
        out_specs=pl.BlockSpec((R, DIM), lambda i: (i, 0)),
        out_shape=jax.ShapeDtypeStruct((N_NODES, DIM), jnp.float32),
    )(x, a, a, w1t, w2t)


def kernel(x, edge_index, W1, W2):
    src = edge_index[0].astype(jnp.int32)
    dst = edge_index[1].astype(jnp.int32)
    pad = EPAD - N_EDGES
    src_f = jnp.concatenate([src, jnp.zeros((pad,), jnp.int32)])
    dst_f = jnp.concatenate([dst, jnp.full((pad,), N_NODES, jnp.int32)])
    s0 = src_f[:E0].reshape(NS, K0, CH)
    s1 = jnp.concatenate(
        [src_f[E0:].reshape(NS, K1, CH),
         jnp.zeros((NS, K0 - K1, CH), jnp.int32)], axis=1)
    src_p = jnp.stack([s0, s1], axis=1).reshape(NW, K0, CH)
    d0 = dst_f[:E0].reshape(NS, K0, CH)
    d1 = jnp.concatenate(
        [dst_f[E0:].reshape(NS, K1, CH),
         jnp.full((NS, K0 - K1, CH), N_NODES, jnp.int32)], axis=1)
    dst_p = jnp.stack([d0, d1], axis=1).reshape(NW, K0, CH)
    zrows = jnp.zeros((ZR, DIM), jnp.float32)
    a = _sc_segment_sum(x, src_p, dst_p, zrows)
    return _tc_finish(x, a, W1.T, W2.T)


# speedup vs baseline: 2.0804x; 1.0473x over previous
import functools

import jax
import jax.numpy as jnp
from jax import lax
from jax.experimental import pallas as pl
from jax.experimental.pallas import tpu as pltpu
from jax.experimental.pallas import tpu_sc as plsc

N_NODES = 10000
N_EDGES = 320000
DIM = 128

NC = 2
NS = 16
NW = NC * NS
CH = 128
K0 = 99
K1 = 58
E0 = NS * K0 * CH
E1 = NS * K1 * CH
EPAD = E0 + E1
ZR = -(-(N_NODES + 1) // (NS * 8)) * 8
A_ROWS = ZR * NS


def _sc_segment_sum(x, src, dst, zrows):
    mesh = plsc.VectorSubcoreMesh(core_axis_name="c", subcore_axis_name="s")

    @functools.partial(
        pl.kernel,
        mesh=mesh,
        out_type=jax.ShapeDtypeStruct((NC, A_ROWS, DIM), jnp.float32),
        scratch_types=[
            pltpu.VMEM((K0, CH), jnp.int32),
            pltpu.VMEM((K0, CH), jnp.int32),
            pltpu.VMEM((CH, DIM), jnp.float32),
            pltpu.VMEM_SHARED((A_ROWS, DIM), jnp.float32),
        ],
    )
    def body(x_hbm, src_hbm, dst_hbm, z_hbm, out_hbm, src_v, dst_v, rows_v, acc):
        c = lax.axis_index("c")
        s = lax.axis_index("s")
        wid = c * NS + s

        pltpu.sync_copy(z_hbm, acc.at[pl.ds(s * ZR, ZR)])
        pltpu.sync_copy(src_hbm.at[wid], src_v)
        pltpu.sync_copy(dst_hbm.at[wid], dst_v)
        plsc.subcore_barrier()

        def step(j, _):
            pltpu.sync_copy(x_hbm.at[src_v.at[j]], rows_v)
            pltpu.sync_copy(rows_v, acc.at[dst_v.at[j]], add=True)
            return _

        kc = jnp.where(c == 0, K0, K1)
        lax.fori_loop(0, kc, step, None)
        plsc.subcore_barrier()
        pltpu.sync_copy(acc.at[pl.ds(s * ZR, ZR)],
                        out_hbm.at[c, pl.ds(s * ZR, ZR)])

    return body(x, src, dst, zrows)


def _tc_finish(x, a, w1t, w2t):
    R = 1000

    def body(x_ref, a0_ref, a1_ref, w1t_ref, w2t_ref, o_ref):
        sp = jnp.dot(x_ref[...], w1t_ref[...],
                     preferred_element_type=jnp.float32,
                     precision=lax.Precision.HIGHEST)
        np_ = jnp.dot(a0_ref[0] + a1_ref[0], w2t_ref[...],
                      preferred_element_type=jnp.float32,
                      precision=lax.Precision.HIGHEST)
        o_ref[...] = jnp.maximum(sp + np_, 0.0)

    return pl.pallas_call(
        body,
        grid=(N_NODES // R,),
        in_specs=[
            pl.BlockSpec((R, DIM), lambda i: (i, 0)),
            pl.BlockSpec((1, R, DIM), lambda i: (0, i, 0)),
            pl.BlockSpec((1, R, DIM), lambda i: (1, i, 0)),
            pl.BlockSpec((DIM, DIM), lambda i: (0, 0)),
            pl.BlockSpec((DIM, DIM), lambda i: (0, 0)),
        ],
        out_specs=pl.BlockSpec((R, DIM), lambda i: (i, 0)),
        out_shape=jax.ShapeDtypeStruct((N_NODES, DIM), jnp.float32),
    )(x, a, a, w1t, w2t)


def kernel(x, edge_index, W1, W2):
    src = edge_index[0].astype(jnp.int32)
    dst = edge_index[1].astype(jnp.int32)
    pad = EPAD - N_EDGES
    src_f = jnp.concatenate([src, jnp.zeros((pad,), jnp.int32)])
    dst_f = jnp.concatenate([dst, jnp.full((pad,), N_NODES, jnp.int32)])
    src_p = jnp.concatenate([
        src_f[:E0].reshape(NS, K0, CH),
        jnp.concatenate([src_f[E0:].reshape(NS, K1, CH),
                         jnp.zeros((NS, K0 - K1, CH), jnp.int32)], axis=1),
    ], axis=0)
    dst_p = jnp.concatenate([
        dst_f[:E0].reshape(NS, K0, CH),
        jnp.concatenate([dst_f[E0:].reshape(NS, K1, CH),
                         jnp.zeros((NS, K0 - K1, CH), jnp.int32)], axis=1),
    ], axis=0)
    zrows = jnp.zeros((ZR, DIM), jnp.float32)
    a = _sc_segment_sum(x, src_p, dst_p, zrows)
    return _tc_finish(x, a, W1.T, W2.T)


# speedup vs baseline: 2.1013x; 1.0100x over previous
import functools

import jax
import jax.numpy as jnp
from jax import lax
from jax.experimental import pallas as pl
from jax.experimental.pallas import tpu as pltpu
from jax.experimental.pallas import tpu_sc as plsc

N_NODES = 10000
N_EDGES = 320000
DIM = 128

NC = 2
NS = 16
NW = NC * NS
CH = 128
K0 = 97
K1 = 60
E0 = NS * K0 * CH
E1 = NS * K1 * CH
EPAD = E0 + E1
ZR = -(-(N_NODES + 1) // (NS * 8)) * 8
A_ROWS = ZR * NS


def _sc_segment_sum(x, src, dst, zrows):
    mesh = plsc.VectorSubcoreMesh(core_axis_name="c", subcore_axis_name="s")

    @functools.partial(
        pl.kernel,
        mesh=mesh,
        out_type=jax.ShapeDtypeStruct((NC, A_ROWS, DIM), jnp.float32),
        scratch_types=[
            pltpu.VMEM((K0, CH), jnp.int32),
            pltpu.VMEM((K0, CH), jnp.int32),
            pltpu.VMEM((CH, DIM), jnp.float32),
            pltpu.VMEM_SHARED((A_ROWS, DIM), jnp.float32),
        ],
    )
    def body(x_hbm, src_hbm, dst_hbm, z_hbm, out_hbm, src_v, dst_v, rows_v, acc):
        c = lax.axis_index("c")
        s = lax.axis_index("s")
        wid = c * NS + s

        pltpu.sync_copy(z_hbm, acc.at[pl.ds(s * ZR, ZR)])
        pltpu.sync_copy(src_hbm.at[wid], src_v)
        pltpu.sync_copy(dst_hbm.at[wid], dst_v)
        plsc.subcore_barrier()

        def step(j, _):
            pltpu.sync_copy(x_hbm.at[src_v.at[j]], rows_v)
            pltpu.sync_copy(rows_v, acc.at[dst_v.at[j]], add=True)
            return _

        kc = jnp.where(c == 0, K0, K1)
        lax.fori_loop(0, kc, step, None)
        plsc.subcore_barrier()
        pltpu.sync_copy(acc.at[pl.ds(s * ZR, ZR)],
                        out_hbm.at[c, pl.ds(s * ZR, ZR)])

    return body(x, src, dst, zrows)


def _tc_finish(x, a, w1t, w2t):
    R = 1000

    def body(x_ref, a0_ref, a1_ref, w1t_ref, w2t_ref, o_ref):
        sp = jnp.dot(x_ref[...], w1t_ref[...],
                     preferred_element_type=jnp.float32,
                     precision=lax.Precision.HIGHEST)
        np_ = jnp.dot(a0_ref[0] + a1_ref[0], w2t_ref[...],
                      preferred_element_type=jnp.float32,
                      precision=lax.Precision.HIGHEST)
        o_ref[...] = jnp.maximum(sp + np_, 0.0)

    return pl.pallas_call(
        body,
        grid=(N_NODES // R,),
        in_specs=[
            pl.BlockSpec((R, DIM), lambda i: (i, 0)),
            pl.BlockSpec((1, R, DIM), lambda i: (0, i, 0)),
            pl.BlockSpec((1, R, DIM), lambda i: (1, i, 0)),
            pl.BlockSpec((DIM, DIM), lambda i: (0, 0)),
            pl.BlockSpec((DIM, DIM), lambda i: (0, 0)),
        ],
        out_specs=pl.BlockSpec((R, DIM), lambda i: (i, 0)),
        out_shape=jax.ShapeDtypeStruct((N_NODES, DIM), jnp.float32),
    )(x, a, a, w1t, w2t)


def kernel(x, edge_index, W1, W2):
    src = edge_index[0].astype(jnp.int32)
    dst = edge_index[1].astype(jnp.int32)
    pad = EPAD - N_EDGES
    src_f = jnp.concatenate([src, jnp.zeros((pad,), jnp.int32)])
    dst_f = jnp.concatenate([dst, jnp.full((pad,), N_NODES, jnp.int32)])
    src_p = jnp.concatenate([
        src_f[:E0].reshape(NS, K0, CH),
        jnp.concatenate([src_f[E0:].reshape(NS, K1, CH),
                         jnp.zeros((NS, K0 - K1, CH), jnp.int32)], axis=1),
    ], axis=0)
    dst_p = jnp.concatenate([
        dst_f[:E0].reshape(NS, K0, CH),
        jnp.concatenate([dst_f[E0:].reshape(NS, K1, CH),
                         jnp.zeros((NS, K0 - K1, CH), jnp.int32)], axis=1),
    ], axis=0)
    zrows = jnp.zeros((ZR, DIM), jnp.float32)
    a = _sc_segment_sum(x, src_p, dst_p, zrows)
    return _tc_finish(x, a, W1.T, W2.T)


# speedup vs baseline: 2.1082x; 1.0033x over previous
import functools

import jax
import jax.numpy as jnp
from jax import lax
from jax.experimental import pallas as pl
from jax.experimental.pallas import tpu as pltpu
from jax.experimental.pallas import tpu_sc as plsc

N_NODES = 10000
N_EDGES = 320000
DIM = 128

NC = 2
NS = 16
NW = NC * NS
CH = 128
K0 = 95
K1 = 62
E0 = NS * K0 * CH
E1 = NS * K1 * CH
EPAD = E0 + E1
ZR = -(-(N_NODES + 1) // (NS * 8)) * 8
A_ROWS = ZR * NS


def _sc_segment_sum(x, src, dst, zrows):
    mesh = plsc.VectorSubcoreMesh(core_axis_name="c", subcore_axis_name="s")

    @functools.partial(
        pl.kernel,
        mesh=mesh,
        out_type=jax.ShapeDtypeStruct((NC, A_ROWS, DIM), jnp.float32),
        scratch_types=[
            pltpu.VMEM((K0, CH), jnp.int32),
            pltpu.VMEM((K0, CH), jnp.int32),
            pltpu.VMEM((CH, DIM), jnp.float32),
            pltpu.VMEM_SHARED((A_ROWS, DIM), jnp.float32),
        ],
    )
    def body(x_hbm, src_hbm, dst_hbm, z_hbm, out_hbm, src_v, dst_v, rows_v, acc):
        c = lax.axis_index("c")
        s = lax.axis_index("s")
        wid = c * NS + s

        pltpu.sync_copy(z_hbm, acc.at[pl.ds(s * ZR, ZR)])
        pltpu.sync_copy(src_hbm.at[wid], src_v)
        pltpu.sync_copy(dst_hbm.at[wid], dst_v)
        plsc.subcore_barrier()

        def step(j, _):
            pltpu.sync_copy(x_hbm.at[src_v.at[j]], rows_v)
            pltpu.sync_copy(rows_v, acc.at[dst_v.at[j]], add=True)
            return _

        kc = jnp.where(c == 0, K0, K1)
        lax.fori_loop(0, kc, step, None)
        plsc.subcore_barrier()
        pltpu.sync_copy(acc.at[pl.ds(s * ZR, ZR)],
                        out_hbm.at[c, pl.ds(s * ZR, ZR)])

    return body(x, src, dst, zrows)


def _tc_finish(x, a, w1t, w2t):
    R = 1000

    def body(x_ref, a0_ref, a1_ref, w1t_ref, w2t_ref, o_ref):
        sp = jnp.dot(x_ref[...], w1t_ref[...],
                     preferred_element_type=jnp.float32,
                     precision=lax.Precision.HIGHEST)
        np_ = jnp.dot(a0_ref[0] + a1_ref[0], w2t_ref[...],
                      preferred_element_type=jnp.float32,
                      precision=lax.Precision.HIGHEST)
        o_ref[...] = jnp.maximum(sp + np_, 0.0)

    return pl.pallas_call(
        body,
        grid=(N_NODES // R,),
        in_specs=[
            pl.BlockSpec((R, DIM), lambda i: (i, 0)),
            pl.BlockSpec((1, R, DIM), lambda i: (0, i, 0)),
            pl.BlockSpec((1, R, DIM), lambda i: (1, i, 0)),
            pl.BlockSpec((DIM, DIM), lambda i: (0, 0)),
            pl.BlockSpec((DIM, DIM), lambda i: (0, 0)),
        ],
        out_specs=pl.BlockSpec((R, DIM), lambda i: (i, 0)),
        out_shape=jax.ShapeDtypeStruct((N_NODES, DIM), jnp.float32),
    )(x, a, a, w1t, w2t)


def kernel(x, edge_index, W1, W2):
    src = edge_index[0].astype(jnp.int32)
    dst = edge_index[1].astype(jnp.int32)
    pad = EPAD - N_EDGES
    src_f = jnp.concatenate([src, jnp.zeros((pad,), jnp.int32)])
    dst_f = jnp.concatenate([dst, jnp.full((pad,), N_NODES, jnp.int32)])
    src_p = jnp.concatenate([
        src_f[:E0].reshape(NS, K0, CH),
        jnp.concatenate([src_f[E0:].reshape(NS, K1, CH),
                         jnp.zeros((NS, K0 - K1, CH), jnp.int32)], axis=1),
    ], axis=0)
    dst_p = jnp.concatenate([
        dst_f[:E0].reshape(NS, K0, CH),
        jnp.concatenate([dst_f[E0:].reshape(NS, K1, CH),
                         jnp.zeros((NS, K0 - K1, CH), jnp.int32)], axis=1),
    ], axis=0)
    zrows = jnp.zeros((ZR, DIM), jnp.float32)
    a = _sc_segment_sum(x, src_p, dst_p, zrows)
    return _tc_finish(x, a, W1.T, W2.T)
